# baseline (speedup 1.0000x reference)
import functools

import jax
import jax.numpy as jnp
from jax import lax
from jax.experimental import pallas as pl
from jax.experimental.pallas import tpu as pltpu
from jax.experimental.pallas import tpu_sc as plsc

N = 2560
E = 40960
HEADS = 4
E2 = E + N
NC, NS, L = 2, 16, 16
EPA = E // (NC * NS)
R = N // NS
B = 2560
NB = E2 // B

_mesh = plsc.VectorSubcoreMesh(
    core_axis_name="c", subcore_axis_name="s", num_cores=NC, num_subcores=NS)

_f32 = jnp.float32
_i32 = jnp.int32


@functools.partial(
    pl.kernel,
    out_type=jax.ShapeDtypeStruct((4 * E,), _f32),
    mesh=_mesh,
    compiler_params=pltpu.CompilerParams(needs_layout_passes=False),
    scratch_types=[
        pltpu.VMEM((N,), _f32),
        pltpu.VMEM((N,), _f32),
        pltpu.VMEM((EPA,), _i32),
        pltpu.VMEM((EPA,), _i32),
        pltpu.VMEM((4, EPA), _f32),
    ],
)
def _sc_gather_pos(px_hbm, py_hbm, src_hbm, dst_hbm, out_hbm, px, py, sv, dv, ob):
    cid = lax.axis_index("c")
    sid = lax.axis_index("s")
    base = (cid * NS + sid) * EPA
    pltpu.sync_copy(px_hbm, px)
    pltpu.sync_copy(py_hbm, py)
    pltpu.sync_copy(src_hbm.at[pl.ds(base, EPA)], sv)
    pltpu.sync_copy(dst_hbm.at[pl.ds(base, EPA)], dv)

    def body(i, _):
        s16 = sv[pl.ds(i * L, L)]
        d16 = dv[pl.ds(i * L, L)]
        ob[0, pl.ds(i * L, L)] = plsc.load_gather(px, [s16])
        ob[1, pl.ds(i * L, L)] = plsc.load_gather(py, [s16])
        ob[2, pl.ds(i * L, L)] = plsc.load_gather(px, [d16])
        ob[3, pl.ds(i * L, L)] = plsc.load_gather(py, [d16])
        return 0

    lax.fori_loop(0, EPA // L, body, 0)
    for r in range(4):
        pltpu.sync_copy(ob.at[r], out_hbm.at[pl.ds(r * E + base, EPA)])


@functools.partial(
    pl.kernel,
    out_type=jax.ShapeDtypeStruct((2 * N,), _f32),
    mesh=_mesh,
    compiler_params=pltpu.CompilerParams(needs_layout_passes=False),
    scratch_types=[
        pltpu.VMEM((R,), _f32),
        pltpu.VMEM((E,), _i32),
        pltpu.VMEM((E,), _f32),
    ],
)
def _sc_scatter_da(da_hbm, src_hbm, out_hbm, acc, sv, vv):
    cid = lax.axis_index("c")
    sid = lax.axis_index("s")
    lo = sid * R
    z = jnp.zeros((L,), _f32)

    def zrow(r, _):
        acc[pl.ds(r * L, L)] = z
        return 0

    lax.fori_loop(0, R // L, zrow, 0)
    pltpu.sync_copy(src_hbm, sv)
    pltpu.sync_copy(da_hbm.at[pl.ds(cid * E, E)], vv)

    def body(i, _):
        s16 = sv[pl.ds(i * L, L)]
        v16 = vv[pl.ds(i * L, L)]
        sl = s16 - lo
        mk = (sl >= 0) & (sl < R)
        slc = jnp.clip(sl, 0, R - 1)
        plsc.addupdate_scatter(acc, [slc], v16, mask=mk)
        return 0

    lax.fori_loop(0, E // L, body, 0)
    pltpu.sync_copy(acc, out_hbm.at[pl.ds(cid * N + lo, R)])


@functools.partial(
    pl.kernel,
    out_type=[
        jax.ShapeDtypeStruct((N, 2 * 512), _f32),
        jax.ShapeDtypeStruct((HEADS * N,), _f32),
    ],
    mesh=_mesh,
    compiler_params=pltpu.CompilerParams(needs_layout_passes=False),
    scratch_types=[
        pltpu.VMEM((R, 512), _f32),
        pltpu.VMEM((N,), _f32),
        pltpu.VMEM((N,), _f32),
        pltpu.VMEM((N,), _f32),
        pltpu.VMEM((N,), _f32),
        pltpu.VMEM((B,), _i32),
        pltpu.VMEM((B,), _i32),
        pltpu.VMEM((B + L,), _i32),
        pltpu.VMEM((B + L,), _i32),
        pltpu.VMEM((B + L,), _f32),
        pltpu.VMEM((B + L,), _f32),
        pltpu.VMEM((L, 512), _f32),
        pltpu.VMEM((L, 512), _f32),
        pltpu.VMEM((R,), _f32),
        pltpu.VMEM((R,), _f32),
        pltpu.SemaphoreType.DMA,
        pltpu.SemaphoreType.DMA,
    ],
)
def _sc_gat_aggregate(h_hbm, alt_hbm, s_hbm, t_hbm, oraw_hbm, ss_hbm,
                      acc, as0, as1, ad0, ad1, sblk, tblk,
                      slist, tlist, elist0, elist1, rows0, rows1,
                      ss0, ss1, semg0, semg1):
    cid = lax.axis_index("c")
    sid = lax.axis_index("s")
    lo = sid * R
    z = jnp.zeros((L,), _f32)

    pltpu.sync_copy(alt_hbm.at[pl.ds((cid * 2) * N, N)], as0)
    pltpu.sync_copy(alt_hbm.at[pl.ds((cid * 2 + 1) * N, N)], as1)
    pltpu.sync_copy(alt_hbm.at[pl.ds((4 + cid * 2) * N, N)], ad0)
    pltpu.sync_copy(alt_hbm.at[pl.ds((5 + cid * 2) * N, N)], ad1)

    def zacc(r, _):
        for m in range(32):
            acc[r, pl.ds(m * L, L)] = z
        return 0

    lax.fori_loop(0, R, zacc, 0)

    def zss(r, _):
        ss0[pl.ds(r * L, L)] = z
        ss1[pl.ds(r * L, L)] = z
        return 0

    lax.fori_loop(0, R // L, zss, 0)
    zi = jnp.zeros((L,), _i32)
    slist[pl.ds(0, L)] = zi
    tlist[pl.ds(0, L)] = zi

    bufs = ((rows0, semg0), (rows1, semg1))

    def block(b, _):
        pltpu.sync_copy(s_hbm.at[pl.ds(b * B, B)], sblk)
        pltpu.sync_copy(t_hbm.at[pl.ds(b * B, B)], tblk)

        def chunk(i, cnt):
            t16 = tblk[pl.ds(i * L, L)]
            tl = t16 - lo
            mk = (tl >= 0) & (tl < R)
            pc = plsc.all_reduce_population_count(mk)[0]

            @pl.when(pc > 0)
            def _():
                s16 = sblk[pl.ds(i * L, L)]
                tlc = jnp.clip(tl, 0, R - 1)
                for hl, (ast, adt, sst, elst) in enumerate(
                        ((as0, ad0, ss0, elist0), (as1, ad1, ss1, elist1))):
                    asv = plsc.load_gather(ast, [s16])
                    adv = plsc.load_gather(adt, [t16])
                    al = asv + adv
                    al = jnp.where(al >= 0.0, al, al * jnp.float32(0.2))
                    ev = jnp.exp(al)
                    plsc.addupdate_scatter(sst, [tlc], ev, mask=mk)
                    plsc.store_compressed(elst.at[pl.ds(cnt, L)], ev, mask=mk)
                plsc.store_compressed(slist.at[pl.ds(cnt, L)], s16, mask=mk)
                plsc.store_compressed(tlist.at[pl.ds(cnt, L)], tlc, mask=mk)

            return cnt + pc

        cnt = lax.fori_loop(0, B // L, chunk, jnp.int32(0))
        elist0[pl.ds(cnt, L)] = z
        elist1[pl.ds(cnt, L)] = z
        zi16 = jnp.zeros((L,), _i32)
        slist[pl.ds(cnt, L)] = zi16
        tlist[pl.ds(cnt, L)] = zi16

        ng = lax.shift_right_logical(cnt + (L - 1), 4)

        def issue(g, rbuf, sem):
            pltpu.async_copy(
                h_hbm.at[cid].at[slist.at[pl.ds(g * L, L)]], rbuf, sem)

        @pl.when(ng >= 1)
        def _():
            issue(jnp.int32(0), rows0, semg0)

        @pl.when(ng >= 2)
        def _():
            issue(jnp.int32(1), rows1, semg1)

        def drain_one(g, rbuf, semg):
            pltpu.make_async_copy(
                h_hbm.at[cid].at[slist.at[pl.ds(g * L, L)]], rbuf, semg).wait()

            tload = tlist[pl.ds(g * L, L)]
            for j in range(L):
                tj = tload[j]
                jj = jnp.full((L,), 0, _i32) + (g * L + j)
                e0j = plsc.load_gather(elist0, [jj])
                e1j = plsc.load_gather(elist1, [jj])
                for m in range(32):
                    ej = e0j if m < 16 else e1j
                    plsc.addupdate(acc.at[tj, pl.ds(m * L, L)],
                                   rbuf[j, pl.ds(m * L, L)] * ej)

            @pl.when(g + 2 < ng)
            def _():
                issue(g + 2, rbuf, semg)

        def pair(i, _):
            for bi in range(2):
                g = i * 2 + bi
                rbuf, semg = bufs[bi]

                @pl.when(g < ng)
                def _():
                    drain_one(g, rbuf, semg)

            return 0

        lax.fori_loop(0, lax.shift_right_logical(ng + 1, 1), pair, 0)
        return 0

    lax.fori_loop(0, NB, block, 0)

    pltpu.sync_copy(
        acc, oraw_hbm.at[pl.ds(lo, R), pl.ds(cid * 512, 512)])
    pltpu.sync_copy(ss0, ss_hbm.at[pl.ds((cid * 2) * N + lo, R)])
    pltpu.sync_copy(ss1, ss_hbm.at[pl.ds((cid * 2 + 1) * N + lo, R)])


def _tc_dist_angle(pg):
    def body(p_ref, o_ref):
        psx = p_ref[0, :]
        psy = p_ref[1, :]
        pdx = p_ref[2, :]
        pdy = p_ref[3, :]
        dx = psx - pdx
        dy = psy - pdy
        o_ref[0, :] = jnp.sqrt(dx * dx + dy * dy)
        o_ref[1, :] = jnp.arctan2(pdy - psy, pdx - psx)

    return pl.pallas_call(
        body, out_shape=jax.ShapeDtypeStruct((2, E), _f32))(pg)


def _tc_mm_alpha(xf, W, Asm):
    def body(x_ref, w_ref, a_ref, h_ref, al_ref):
        h = jnp.dot(x_ref[...], w_ref[...], preferred_element_type=_f32)
        h_ref[0] = h[:, :512]
        h_ref[1] = h[:, 512:]
        al_ref[...] = jnp.dot(h, a_ref[...], preferred_element_type=_f32)

    return pl.pallas_call(
        body,
        out_shape=[
            jax.ShapeDtypeStruct((2, N, 512), _f32),
            jax.ShapeDtypeStruct((N, 8), _f32),
        ])(xf, W, Asm)


def _epilogue(oraw_ref, ss_ref, b, g, be):
    zs = []
    for k in range(4):
        den = ss_ref[k, :][:, None] + jnp.float32(1e-16)
        zs.append(oraw_ref[:, k * 256:(k + 1) * 256] / den)
    zcat = jnp.concatenate(zs, axis=1) + b[...][None, :]
    mu = jnp.mean(zcat, axis=0, keepdims=True)
    d = zcat - mu
    v = jnp.mean(d * d, axis=0, keepdims=True)
    zn = d / jnp.sqrt(v + jnp.float32(1e-5)) * g[...][None, :] + be[...][None, :]
    return jnp.maximum(zn, 0.0)


def _tc_epi_mm_alpha(oraw, ssT, b, g, be, W, Asm):
    def body(o_ref, s_ref, b_ref, g_ref, be_ref, w_ref, a_ref, h_ref, al_ref):
        h = _epilogue(o_ref, s_ref, b_ref, g_ref, be_ref)
        h2 = jnp.dot(h, w_ref[...], preferred_element_type=_f32)
        h_ref[0] = h2[:, :512]
        h_ref[1] = h2[:, 512:]
        al_ref[...] = jnp.dot(h2, a_ref[...], preferred_element_type=_f32)

    return pl.pallas_call(
        body,
        out_shape=[
            jax.ShapeDtypeStruct((2, N, 512), _f32),
            jax.ShapeDtypeStruct((N, 8), _f32),
        ])(oraw, ssT, b, g, be, W, Asm)


def _tc_final(oraw, ssT, b, g, be, fcW, fcb, mask, pos):
    def body(o_ref, s_ref, b_ref, g_ref, be_ref, w_ref, fb_ref, m_ref, p_ref,
             on_ref, up_ref):
        h = _epilogue(o_ref, s_ref, b_ref, g_ref, be_ref)
        on = jnp.dot(h, w_ref[...], preferred_element_type=_f32) + fb_ref[...][None, :]
        on = on * m_ref[...][:, None]
        on_ref[...] = on
        up_ref[...] = p_ref[...] + on

    return pl.pallas_call(
        body,
        out_shape=[
            jax.ShapeDtypeStruct((N, 2), _f32),
            jax.ShapeDtypeStruct((N, 2), _f32),
        ])(oraw, ssT, b, g, be, fcW, fcb, mask, pos)


def _block_diag_alpha(a_s, a_d):
    eye = jnp.eye(4, dtype=_f32)
    bs = jnp.einsum("kc,kj->kcj", a_s, eye).reshape(1024, 4)
    bd = jnp.einsum("kc,kj->kcj", a_d, eye).reshape(1024, 4)
    return jnp.concatenate([bs, bd], axis=1)


def kernel(x, edge_index, pos, mask, batch, W1, a_src1, a_dst1, b1, g1, be1,
           W2, a_src2, a_dst2, b2, g2, be2, fcW, fcb):
    src0 = edge_index[0]
    dst0 = edge_index[1]
    px = pos[:, 0]
    py = pos[:, 1]

    pg = _sc_gather_pos(px, py, src0, dst0).reshape(4, E)
    da = _tc_dist_angle(pg).reshape(2 * E)
    dsum_asum = _sc_scatter_da(da, src0).reshape(2, N)
    xf = jnp.concatenate([x, dsum_asum[0][:, None], dsum_asum[1][:, None]],
                         axis=1)

    loop = jnp.arange(N, dtype=src0.dtype)
    s2 = jnp.concatenate([src0, loop])
    t2 = jnp.concatenate([dst0, loop])

    Asm1 = _block_diag_alpha(a_src1, a_dst1)
    Asm2 = _block_diag_alpha(a_src2, a_dst2)

    h1, al1 = _tc_mm_alpha(xf, W1, Asm1)
    oraw1, ss1 = _sc_gat_aggregate(h1, al1.T.reshape(8 * N), s2, t2)

    h2, al2 = _tc_epi_mm_alpha(oraw1, ss1.reshape(4, N), b1, g1, be1, W2, Asm2)
    oraw2, ss2 = _sc_gat_aggregate(h2, al2.T.reshape(8 * N), s2, t2)

    on, up = _tc_final(oraw2, ss2.reshape(4, N), b2, g2, be2, fcW, fcb, mask, pos)
    return (on.reshape(64, 40, 2), up.reshape(64, 40, 2))


# speedup vs baseline: 1.3890x; 1.3890x over previous
import functools

import jax
import jax.numpy as jnp
from jax import lax
from jax.experimental import pallas as pl
from jax.experimental.pallas import tpu as pltpu
from jax.experimental.pallas import tpu_sc as plsc

N = 2560
E = 40960
HEADS = 4
E2 = E + N
NC, NS, L = 2, 16, 16
EPA = E // (NC * NS)
R = N // NS
B = 2560
NB = E2 // B

_mesh = plsc.VectorSubcoreMesh(
    core_axis_name="c", subcore_axis_name="s", num_cores=NC, num_subcores=NS)

_f32 = jnp.float32
_i32 = jnp.int32


@functools.partial(
    pl.kernel,
    out_type=jax.ShapeDtypeStruct((4 * E,), _f32),
    mesh=_mesh,
    compiler_params=pltpu.CompilerParams(needs_layout_passes=False),
    scratch_types=[
        pltpu.VMEM((N,), _f32),
        pltpu.VMEM((N,), _f32),
        pltpu.VMEM((EPA,), _i32),
        pltpu.VMEM((EPA,), _i32),
        pltpu.VMEM((4, EPA), _f32),
    ],
)
def _sc_gather_pos(px_hbm, py_hbm, src_hbm, dst_hbm, out_hbm, px, py, sv, dv, ob):
    cid = lax.axis_index("c")
    sid = lax.axis_index("s")
    base = (cid * NS + sid) * EPA
    pltpu.sync_copy(px_hbm, px)
    pltpu.sync_copy(py_hbm, py)
    pltpu.sync_copy(src_hbm.at[pl.ds(base, EPA)], sv)
    pltpu.sync_copy(dst_hbm.at[pl.ds(base, EPA)], dv)

    def body(i, _):
        s16 = sv[pl.ds(i * L, L)]
        d16 = dv[pl.ds(i * L, L)]
        ob[0, pl.ds(i * L, L)] = plsc.load_gather(px, [s16])
        ob[1, pl.ds(i * L, L)] = plsc.load_gather(py, [s16])
        ob[2, pl.ds(i * L, L)] = plsc.load_gather(px, [d16])
        ob[3, pl.ds(i * L, L)] = plsc.load_gather(py, [d16])
        return 0

    lax.fori_loop(0, EPA // L, body, 0)
    for r in range(4):
        pltpu.sync_copy(ob.at[r], out_hbm.at[pl.ds(r * E + base, EPA)])


@functools.partial(
    pl.kernel,
    out_type=jax.ShapeDtypeStruct((2 * N,), _f32),
    mesh=_mesh,
    compiler_params=pltpu.CompilerParams(needs_layout_passes=False),
    scratch_types=[
        pltpu.VMEM((R,), _f32),
        pltpu.VMEM((E,), _i32),
        pltpu.VMEM((E,), _f32),
    ],
)
def _sc_scatter_da(da_hbm, src_hbm, out_hbm, acc, sv, vv):
    cid = lax.axis_index("c")
    sid = lax.axis_index("s")
    lo = sid * R
    z = jnp.zeros((L,), _f32)

    def zrow(r, _):
        acc[pl.ds(r * L, L)] = z
        return 0

    lax.fori_loop(0, R // L, zrow, 0)
    pltpu.sync_copy(src_hbm, sv)
    pltpu.sync_copy(da_hbm.at[pl.ds(cid * E, E)], vv)

    def body(i, _):
        s16 = sv[pl.ds(i * L, L)]
        v16 = vv[pl.ds(i * L, L)]
        sl = s16 - lo
        mk = (sl >= 0) & (sl < R)
        slc = jnp.clip(sl, 0, R - 1)
        plsc.addupdate_scatter(acc, [slc], v16, mask=mk)
        return 0

    lax.fori_loop(0, E // L, body, 0)
    pltpu.sync_copy(acc, out_hbm.at[pl.ds(cid * N + lo, R)])


@functools.partial(
    pl.kernel,
    out_type=[
        jax.ShapeDtypeStruct((N, 2 * 512), _f32),
        jax.ShapeDtypeStruct((HEADS * N,), _f32),
    ],
    mesh=_mesh,
    compiler_params=pltpu.CompilerParams(needs_layout_passes=False),
    scratch_types=[
        pltpu.VMEM((R, 512), _f32),
        pltpu.VMEM((N,), _f32),
        pltpu.VMEM((N,), _f32),
        pltpu.VMEM((N,), _f32),
        pltpu.VMEM((N,), _f32),
        pltpu.VMEM((B,), _i32),
        pltpu.VMEM((B,), _i32),
        pltpu.VMEM((B + L,), _i32),
        pltpu.VMEM((B + L,), _i32),
        pltpu.VMEM((B + L,), _f32),
        pltpu.VMEM((B + L,), _f32),
        pltpu.VMEM((L, 512), _f32),
        pltpu.VMEM((L, 512), _f32),
        pltpu.VMEM((R,), _f32),
        pltpu.VMEM((R,), _f32),
        pltpu.SemaphoreType.DMA,
        pltpu.SemaphoreType.DMA,
    ],
)
def _sc_gat_aggregate(h_hbm, alt_hbm, s_hbm, t_hbm, oraw_hbm, ss_hbm,
                      acc, as0, as1, ad0, ad1, sblk, tblk,
                      slist, tlist, elist0, elist1, rows0, rows1,
                      ss0, ss1, semg0, semg1):
    cid = lax.axis_index("c")
    sid = lax.axis_index("s")
    lo = sid * R
    z = jnp.zeros((L,), _f32)

    pltpu.sync_copy(alt_hbm.at[pl.ds((cid * 2) * N, N)], as0)
    pltpu.sync_copy(alt_hbm.at[pl.ds((cid * 2 + 1) * N, N)], as1)
    pltpu.sync_copy(alt_hbm.at[pl.ds((4 + cid * 2) * N, N)], ad0)
    pltpu.sync_copy(alt_hbm.at[pl.ds((5 + cid * 2) * N, N)], ad1)

    def zacc(r, _):
        for m in range(32):
            acc[r, pl.ds(m * L, L)] = z
        return 0

    lax.fori_loop(0, R, zacc, 0)

    def zss(r, _):
        ss0[pl.ds(r * L, L)] = z
        ss1[pl.ds(r * L, L)] = z
        return 0

    lax.fori_loop(0, R // L, zss, 0)
    zi = jnp.zeros((L,), _i32)
    slist[pl.ds(0, L)] = zi
    tlist[pl.ds(0, L)] = zi

    bufs = ((rows0, semg0), (rows1, semg1))

    def block(b, _):
        pltpu.sync_copy(s_hbm.at[pl.ds(b * B, B)], sblk)
        pltpu.sync_copy(t_hbm.at[pl.ds(b * B, B)], tblk)

        def chunk(i, cnt):
            t16 = tblk[pl.ds(i * L, L)]
            tl = t16 - lo
            mk = (tl >= 0) & (tl < R)
            pc = plsc.all_reduce_population_count(mk)[0]

            @pl.when(pc > 0)
            def _():
                s16 = sblk[pl.ds(i * L, L)]
                tlc = jnp.clip(tl, 0, R - 1)
                for hl, (ast, adt, sst, elst) in enumerate(
                        ((as0, ad0, ss0, elist0), (as1, ad1, ss1, elist1))):
                    asv = plsc.load_gather(ast, [s16])
                    adv = plsc.load_gather(adt, [t16])
                    al = asv + adv
                    al = jnp.where(al >= 0.0, al, al * jnp.float32(0.2))
                    ev = jnp.exp(al)
                    plsc.addupdate_scatter(sst, [tlc], ev, mask=mk)
                    plsc.store_compressed(elst.at[pl.ds(cnt, L)], ev, mask=mk)
                plsc.store_compressed(slist.at[pl.ds(cnt, L)], s16, mask=mk)
                plsc.store_compressed(tlist.at[pl.ds(cnt, L)], tlc, mask=mk)

            return cnt + pc

        cnt = lax.fori_loop(0, B // L, chunk, jnp.int32(0))
        elist0[pl.ds(cnt, L)] = z
        elist1[pl.ds(cnt, L)] = z
        zi16 = jnp.zeros((L,), _i32)
        slist[pl.ds(cnt, L)] = zi16
        tlist[pl.ds(cnt, L)] = zi16

        ng = lax.shift_right_logical(cnt + (L - 1), 4)

        def issue(g, rbuf, sem):
            pltpu.async_copy(
                h_hbm.at[cid].at[slist.at[pl.ds(g * L, L)]], rbuf, sem)

        @pl.when(ng >= 1)
        def _():
            issue(jnp.int32(0), rows0, semg0)

        @pl.when(ng >= 2)
        def _():
            issue(jnp.int32(1), rows1, semg1)

        def drain_one(g, rbuf, semg):
            pltpu.make_async_copy(
                h_hbm.at[cid].at[slist.at[pl.ds(g * L, L)]], rbuf, semg).wait()

            def jbody(j, _):
                jj = jnp.full((L,), 0, _i32) + (g * L + j)
                tj = plsc.load_gather(tlist, [jj])[0]
                e0j = plsc.load_gather(elist0, [jj])
                e1j = plsc.load_gather(elist1, [jj])
                for m in range(32):
                    ej = e0j if m < 16 else e1j
                    plsc.addupdate(acc.at[tj, pl.ds(m * L, L)],
                                   rbuf[j, pl.ds(m * L, L)] * ej)
                return 0

            lax.fori_loop(0, L, jbody, 0)

            @pl.when(g + 2 < ng)
            def _():
                issue(g + 2, rbuf, semg)

        def pair(i, _):
            for bi in range(2):
                g = i * 2 + bi
                rbuf, semg = bufs[bi]

                @pl.when(g < ng)
                def _():
                    drain_one(g, rbuf, semg)

            return 0

        lax.fori_loop(0, lax.shift_right_logical(ng + 1, 1), pair, 0)
        return 0

    lax.fori_loop(0, NB, block, 0)

    pltpu.sync_copy(
        acc, oraw_hbm.at[pl.ds(lo, R), pl.ds(cid * 512, 512)])
    pltpu.sync_copy(ss0, ss_hbm.at[pl.ds((cid * 2) * N + lo, R)])
    pltpu.sync_copy(ss1, ss_hbm.at[pl.ds((cid * 2 + 1) * N + lo, R)])


def _tc_dist_angle(pg):
    def body(p_ref, o_ref):
        psx = p_ref[0, :]
        psy = p_ref[1, :]
        pdx = p_ref[2, :]
        pdy = p_ref[3, :]
        dx = psx - pdx
        dy = psy - pdy
        o_ref[0, :] = jnp.sqrt(dx * dx + dy * dy)
        o_ref[1, :] = jnp.arctan2(pdy - psy, pdx - psx)

    return pl.pallas_call(
        body, out_shape=jax.ShapeDtypeStruct((2, E), _f32))(pg)


def _tc_mm_alpha(xf, W, Asm):
    def body(x_ref, w_ref, a_ref, h_ref, al_ref):
        h = jnp.dot(x_ref[...], w_ref[...], preferred_element_type=_f32)
        h_ref[0] = h[:, :512]
        h_ref[1] = h[:, 512:]
        al_ref[...] = jnp.dot(h, a_ref[...], preferred_element_type=_f32)

    return pl.pallas_call(
        body,
        out_shape=[
            jax.ShapeDtypeStruct((2, N, 512), _f32),
            jax.ShapeDtypeStruct((N, 8), _f32),
        ])(xf, W, Asm)


def _epilogue(oraw_ref, ss_ref, b, g, be):
    zs = []
    for k in range(4):
        den = ss_ref[k, :][:, None] + jnp.float32(1e-16)
        zs.append(oraw_ref[:, k * 256:(k + 1) * 256] / den)
    zcat = jnp.concatenate(zs, axis=1) + b[...][None, :]
    mu = jnp.mean(zcat, axis=0, keepdims=True)
    d = zcat - mu
    v = jnp.mean(d * d, axis=0, keepdims=True)
    zn = d / jnp.sqrt(v + jnp.float32(1e-5)) * g[...][None, :] + be[...][None, :]
    return jnp.maximum(zn, 0.0)


def _tc_epi_mm_alpha(oraw, ssT, b, g, be, W, Asm):
    def body(o_ref, s_ref, b_ref, g_ref, be_ref, w_ref, a_ref, h_ref, al_ref):
        h = _epilogue(o_ref, s_ref, b_ref, g_ref, be_ref)
        h2 = jnp.dot(h, w_ref[...], preferred_element_type=_f32)
        h_ref[0] = h2[:, :512]
        h_ref[1] = h2[:, 512:]
        al_ref[...] = jnp.dot(h2, a_ref[...], preferred_element_type=_f32)

    return pl.pallas_call(
        body,
        out_shape=[
            jax.ShapeDtypeStruct((2, N, 512), _f32),
            jax.ShapeDtypeStruct((N, 8), _f32),
        ])(oraw, ssT, b, g, be, W, Asm)


def _tc_final(oraw, ssT, b, g, be, fcW, fcb, mask, pos):
    def body(o_ref, s_ref, b_ref, g_ref, be_ref, w_ref, fb_ref, m_ref, p_ref,
             on_ref, up_ref):
        h = _epilogue(o_ref, s_ref, b_ref, g_ref, be_ref)
        on = jnp.dot(h, w_ref[...], preferred_element_type=_f32) + fb_ref[...][None, :]
        on = on * m_ref[...][:, None]
        on_ref[...] = on
        up_ref[...] = p_ref[...] + on

    return pl.pallas_call(
        body,
        out_shape=[
            jax.ShapeDtypeStruct((N, 2), _f32),
            jax.ShapeDtypeStruct((N, 2), _f32),
        ])(oraw, ssT, b, g, be, fcW, fcb, mask, pos)


def _block_diag_alpha(a_s, a_d):
    eye = jnp.eye(4, dtype=_f32)
    bs = jnp.einsum("kc,kj->kcj", a_s, eye).reshape(1024, 4)
    bd = jnp.einsum("kc,kj->kcj", a_d, eye).reshape(1024, 4)
    return jnp.concatenate([bs, bd], axis=1)


def kernel(x, edge_index, pos, mask, batch, W1, a_src1, a_dst1, b1, g1, be1,
           W2, a_src2, a_dst2, b2, g2, be2, fcW, fcb):
    src0 = edge_index[0]
    dst0 = edge_index[1]
    px = pos[:, 0]
    py = pos[:, 1]

    pg = _sc_gather_pos(px, py, src0, dst0).reshape(4, E)
    da = _tc_dist_angle(pg).reshape(2 * E)
    dsum_asum = _sc_scatter_da(da, src0).reshape(2, N)
    xf = jnp.concatenate([x, dsum_asum[0][:, None], dsum_asum[1][:, None]],
                         axis=1)

    loop = jnp.arange(N, dtype=src0.dtype)
    s2 = jnp.concatenate([src0, loop])
    t2 = jnp.concatenate([dst0, loop])

    Asm1 = _block_diag_alpha(a_src1, a_dst1)
    Asm2 = _block_diag_alpha(a_src2, a_dst2)

    h1, al1 = _tc_mm_alpha(xf, W1, Asm1)
    oraw1, ss1 = _sc_gat_aggregate(h1, al1.T.reshape(8 * N), s2, t2)

    h2, al2 = _tc_epi_mm_alpha(oraw1, ss1.reshape(4, N), b1, g1, be1, W2, Asm2)
    oraw2, ss2 = _sc_gat_aggregate(h2, al2.T.reshape(8 * N), s2, t2)

    on, up = _tc_final(oraw2, ss2.reshape(4, N), b2, g2, be2, fcW, fcb, mask, pos)
    return (on.reshape(64, 40, 2), up.reshape(64, 40, 2))


# speedup vs baseline: 1.5454x; 1.1126x over previous
import functools

import jax
import jax.numpy as jnp
from jax import lax
from jax.experimental import pallas as pl
from jax.experimental.pallas import tpu as pltpu
from jax.experimental.pallas import tpu_sc as plsc

N = 2560
E = 40960
HEADS = 4
E2 = E + N
NC, NS, L = 2, 16, 16
EPA = E // (NC * NS)
R = N // NS
B = 2560
NB = E2 // B

_mesh = plsc.VectorSubcoreMesh(
    core_axis_name="c", subcore_axis_name="s", num_cores=NC, num_subcores=NS)

_f32 = jnp.float32
_i32 = jnp.int32


@functools.partial(
    pl.kernel,
    out_type=jax.ShapeDtypeStruct((4 * E,), _f32),
    mesh=_mesh,
    compiler_params=pltpu.CompilerParams(needs_layout_passes=False),
    scratch_types=[
        pltpu.VMEM((N,), _f32),
        pltpu.VMEM((N,), _f32),
        pltpu.VMEM((EPA,), _i32),
        pltpu.VMEM((EPA,), _i32),
        pltpu.VMEM((4, EPA), _f32),
    ],
)
def _sc_gather_pos(px_hbm, py_hbm, src_hbm, dst_hbm, out_hbm, px, py, sv, dv, ob):
    cid = lax.axis_index("c")
    sid = lax.axis_index("s")
    base = (cid * NS + sid) * EPA
    pltpu.sync_copy(px_hbm, px)
    pltpu.sync_copy(py_hbm, py)
    pltpu.sync_copy(src_hbm.at[pl.ds(base, EPA)], sv)
    pltpu.sync_copy(dst_hbm.at[pl.ds(base, EPA)], dv)

    def body(i, _):
        s16 = sv[pl.ds(i * L, L)]
        d16 = dv[pl.ds(i * L, L)]
        ob[0, pl.ds(i * L, L)] = plsc.load_gather(px, [s16])
        ob[1, pl.ds(i * L, L)] = plsc.load_gather(py, [s16])
        ob[2, pl.ds(i * L, L)] = plsc.load_gather(px, [d16])
        ob[3, pl.ds(i * L, L)] = plsc.load_gather(py, [d16])
        return 0

    lax.fori_loop(0, EPA // L, body, 0)
    for r in range(4):
        pltpu.sync_copy(ob.at[r], out_hbm.at[pl.ds(r * E + base, EPA)])


@functools.partial(
    pl.kernel,
    out_type=jax.ShapeDtypeStruct((2 * N,), _f32),
    mesh=_mesh,
    compiler_params=pltpu.CompilerParams(needs_layout_passes=False),
    scratch_types=[
        pltpu.VMEM((R,), _f32),
        pltpu.VMEM((E,), _i32),
        pltpu.VMEM((E,), _f32),
    ],
)
def _sc_scatter_da(da_hbm, src_hbm, out_hbm, acc, sv, vv):
    cid = lax.axis_index("c")
    sid = lax.axis_index("s")
    lo = sid * R
    z = jnp.zeros((L,), _f32)

    def zrow(r, _):
        acc[pl.ds(r * L, L)] = z
        return 0

    lax.fori_loop(0, R // L, zrow, 0)
    pltpu.sync_copy(src_hbm, sv)
    pltpu.sync_copy(da_hbm.at[pl.ds(cid * E, E)], vv)

    def body(i, _):
        s16 = sv[pl.ds(i * L, L)]
        v16 = vv[pl.ds(i * L, L)]
        sl = s16 - lo
        mk = (sl >= 0) & (sl < R)
        slc = jnp.clip(sl, 0, R - 1)
        plsc.addupdate_scatter(acc, [slc], v16, mask=mk)
        return 0

    lax.fori_loop(0, E // L, body, 0)
    pltpu.sync_copy(acc, out_hbm.at[pl.ds(cid * N + lo, R)])


@functools.partial(
    pl.kernel,
    out_type=[
        jax.ShapeDtypeStruct((N, 2 * 512), _f32),
        jax.ShapeDtypeStruct((HEADS * N,), _f32),
    ],
    mesh=_mesh,
    compiler_params=pltpu.CompilerParams(needs_layout_passes=False),
    scratch_types=[
        pltpu.VMEM((R, 512), _f32),
        pltpu.VMEM((N,), _f32),
        pltpu.VMEM((N,), _f32),
        pltpu.VMEM((N,), _f32),
        pltpu.VMEM((N,), _f32),
        pltpu.VMEM((B,), _i32),
        pltpu.VMEM((B,), _i32),
        pltpu.VMEM((B + L,), _i32),
        pltpu.VMEM((B + L,), _i32),
        pltpu.VMEM((2 * L,), _f32),
        pltpu.VMEM((L, 512), _f32),
        pltpu.VMEM((L, 512), _f32),
        pltpu.VMEM((R,), _f32),
        pltpu.VMEM((R,), _f32),
        pltpu.SemaphoreType.DMA,
        pltpu.SemaphoreType.DMA,
    ],
)
def _sc_gat_aggregate(h_hbm, alt_hbm, s_hbm, t_hbm, oraw_hbm, ss_hbm,
                      acc, as0, as1, ad0, ad1, sblk, tblk,
                      slist, tlist, ebuf, rows0, rows1,
                      ss0, ss1, semg0, semg1):
    cid = lax.axis_index("c")
    sid = lax.axis_index("s")
    lo = sid * R
    z = jnp.zeros((L,), _f32)

    pltpu.sync_copy(alt_hbm.at[pl.ds((cid * 2) * N, N)], as0)
    pltpu.sync_copy(alt_hbm.at[pl.ds((cid * 2 + 1) * N, N)], as1)
    pltpu.sync_copy(alt_hbm.at[pl.ds((4 + cid * 2) * N, N)], ad0)
    pltpu.sync_copy(alt_hbm.at[pl.ds((5 + cid * 2) * N, N)], ad1)

    def zacc(r, _):
        for m in range(32):
            acc[r, pl.ds(m * L, L)] = z
        return 0

    lax.fori_loop(0, R, zacc, 0)

    def zss(r, _):
        ss0[pl.ds(r * L, L)] = z
        ss1[pl.ds(r * L, L)] = z
        return 0

    lax.fori_loop(0, R // L, zss, 0)
    zi = jnp.zeros((L,), _i32)
    slist[pl.ds(0, L)] = zi
    tlist[pl.ds(0, L)] = zi

    bufs = ((rows0, semg0), (rows1, semg1))

    def block(b, _):
        pltpu.sync_copy(s_hbm.at[pl.ds(b * B, B)], sblk)
        pltpu.sync_copy(t_hbm.at[pl.ds(b * B, B)], tblk)

        def chunk(i, cnt):
            t16 = tblk[pl.ds(i * L, L)]
            tl = t16 - lo
            mk = (tl >= 0) & (tl < R)
            pc = plsc.all_reduce_population_count(mk)[0]

            @pl.when(pc > 0)
            def _():
                s16 = sblk[pl.ds(i * L, L)]
                tlc = jnp.clip(tl, 0, R - 1)
                plsc.store_compressed(slist.at[pl.ds(cnt, L)], s16, mask=mk)
                plsc.store_compressed(tlist.at[pl.ds(cnt, L)], tlc, mask=mk)

            return cnt + pc

        cnt = lax.fori_loop(0, B // L, chunk, jnp.int32(0))
        zi16 = jnp.zeros((L,), _i32)
        slist[pl.ds(cnt, L)] = zi16
        tlist[pl.ds(cnt, L)] = zi16

        ng = lax.shift_right_logical(cnt + (L - 1), 4)

        def issue(g, rbuf, sem):
            pltpu.async_copy(
                h_hbm.at[cid].at[slist.at[pl.ds(g * L, L)]], rbuf, sem)

        @pl.when(ng >= 1)
        def _():
            issue(jnp.int32(0), rows0, semg0)

        @pl.when(ng >= 2)
        def _():
            issue(jnp.int32(1), rows1, semg1)

        def drain_one(g, cnt, rbuf, semg):
            sv16 = slist[pl.ds(g * L, L)]
            tlv = tlist[pl.ds(g * L, L)]
            tgv = tlv + lo
            mkd = (lax.iota(_i32, L) + g * L) < cnt
            for hl, (ast, adt, sst) in enumerate(
                    ((as0, ad0, ss0), (as1, ad1, ss1))):
                asv = plsc.load_gather(ast, [sv16])
                adv = plsc.load_gather(adt, [tgv])
                al = asv + adv
                al = jnp.where(al >= 0.0, al, al * jnp.float32(0.2))
                ev = jnp.exp(al)
                ev = jnp.where(mkd, ev, jnp.float32(0.0))
                plsc.addupdate_scatter(sst, [tlv], ev, mask=mkd)
                ebuf[pl.ds(hl * L, L)] = ev

            pltpu.make_async_copy(
                h_hbm.at[cid].at[slist.at[pl.ds(g * L, L)]], rbuf, semg).wait()

            def jbody(j, _):
                jj = jnp.full((L,), 0, _i32) + j
                tj = plsc.load_gather(tlist, [jj + g * L])[0]
                e0j = plsc.load_gather(ebuf, [jj])
                e1j = plsc.load_gather(ebuf, [jj + L])
                for m in range(32):
                    ej = e0j if m < 16 else e1j
                    plsc.addupdate(acc.at[tj, pl.ds(m * L, L)],
                                   rbuf[j, pl.ds(m * L, L)] * ej)
                return 0

            lax.fori_loop(0, L, jbody, 0)

            @pl.when(g + 2 < ng)
            def _():
                issue(g + 2, rbuf, semg)

        def pair(i, _):
            for bi in range(2):
                g = i * 2 + bi
                rbuf, semg = bufs[bi]

                @pl.when(g < ng)
                def _():
                    drain_one(g, cnt, rbuf, semg)

            return 0

        lax.fori_loop(0, lax.shift_right_logical(ng + 1, 1), pair, 0)
        return 0

    lax.fori_loop(0, NB, block, 0)

    pltpu.sync_copy(
        acc, oraw_hbm.at[pl.ds(lo, R), pl.ds(cid * 512, 512)])
    pltpu.sync_copy(ss0, ss_hbm.at[pl.ds((cid * 2) * N + lo, R)])
    pltpu.sync_copy(ss1, ss_hbm.at[pl.ds((cid * 2 + 1) * N + lo, R)])


def _tc_dist_angle(pg):
    def body(p_ref, o_ref):
        psx = p_ref[0, :]
        psy = p_ref[1, :]
        pdx = p_ref[2, :]
        pdy = p_ref[3, :]
        dx = psx - pdx
        dy = psy - pdy
        o_ref[0, :] = jnp.sqrt(dx * dx + dy * dy)
        o_ref[1, :] = jnp.arctan2(pdy - psy, pdx - psx)

    return pl.pallas_call(
        body, out_shape=jax.ShapeDtypeStruct((2, E), _f32))(pg)


def _tc_mm_alpha(xf, W, Asm):
    def body(x_ref, w_ref, a_ref, h_ref, al_ref):
        h = jnp.dot(x_ref[...], w_ref[...], preferred_element_type=_f32)
        h_ref[0] = h[:, :512]
        h_ref[1] = h[:, 512:]
        al_ref[...] = jnp.dot(h, a_ref[...], preferred_element_type=_f32)

    return pl.pallas_call(
        body,
        out_shape=[
            jax.ShapeDtypeStruct((2, N, 512), _f32),
            jax.ShapeDtypeStruct((N, 8), _f32),
        ])(xf, W, Asm)


def _epilogue(oraw_ref, ss_ref, b, g, be):
    zs = []
    for k in range(4):
        den = ss_ref[k, :][:, None] + jnp.float32(1e-16)
        zs.append(oraw_ref[:, k * 256:(k + 1) * 256] / den)
    zcat = jnp.concatenate(zs, axis=1) + b[...][None, :]
    mu = jnp.mean(zcat, axis=0, keepdims=True)
    d = zcat - mu
    v = jnp.mean(d * d, axis=0, keepdims=True)
    zn = d / jnp.sqrt(v + jnp.float32(1e-5)) * g[...][None, :] + be[...][None, :]
    return jnp.maximum(zn, 0.0)


def _tc_epi_mm_alpha(oraw, ssT, b, g, be, W, Asm):
    def body(o_ref, s_ref, b_ref, g_ref, be_ref, w_ref, a_ref, h_ref, al_ref):
        h = _epilogue(o_ref, s_ref, b_ref, g_ref, be_ref)
        h2 = jnp.dot(h, w_ref[...], preferred_element_type=_f32)
        h_ref[0] = h2[:, :512]
        h_ref[1] = h2[:, 512:]
        al_ref[...] = jnp.dot(h2, a_ref[...], preferred_element_type=_f32)

    return pl.pallas_call(
        body,
        out_shape=[
            jax.ShapeDtypeStruct((2, N, 512), _f32),
            jax.ShapeDtypeStruct((N, 8), _f32),
        ])(oraw, ssT, b, g, be, W, Asm)


def _tc_final(oraw, ssT, b, g, be, fcW, fcb, mask, pos):
    def body(o_ref, s_ref, b_ref, g_ref, be_ref, w_ref, fb_ref, m_ref, p_ref,
             on_ref, up_ref):
        h = _epilogue(o_ref, s_ref, b_ref, g_ref, be_ref)
        on = jnp.dot(h, w_ref[...], preferred_element_type=_f32) + fb_ref[...][None, :]
        on = on * m_ref[...][:, None]
        on_ref[...] = on
        up_ref[...] = p_ref[...] + on

    return pl.pallas_call(
        body,
        out_shape=[
            jax.ShapeDtypeStruct((N, 2), _f32),
            jax.ShapeDtypeStruct((N, 2), _f32),
        ])(oraw, ssT, b, g, be, fcW, fcb, mask, pos)


def _block_diag_alpha(a_s, a_d):
    eye = jnp.eye(4, dtype=_f32)
    bs = jnp.einsum("kc,kj->kcj", a_s, eye).reshape(1024, 4)
    bd = jnp.einsum("kc,kj->kcj", a_d, eye).reshape(1024, 4)
    return jnp.concatenate([bs, bd], axis=1)


def kernel(x, edge_index, pos, mask, batch, W1, a_src1, a_dst1, b1, g1, be1,
           W2, a_src2, a_dst2, b2, g2, be2, fcW, fcb):
    src0 = edge_index[0]
    dst0 = edge_index[1]
    px = pos[:, 0]
    py = pos[:, 1]

    pg = _sc_gather_pos(px, py, src0, dst0).reshape(4, E)
    da = _tc_dist_angle(pg).reshape(2 * E)
    dsum_asum = _sc_scatter_da(da, src0).reshape(2, N)
    xf = jnp.concatenate([x, dsum_asum[0][:, None], dsum_asum[1][:, None]],
                         axis=1)

    loop = jnp.arange(N, dtype=src0.dtype)
    s2 = jnp.concatenate([src0, loop])
    t2 = jnp.concatenate([dst0, loop])

    Asm1 = _block_diag_alpha(a_src1, a_dst1)
    Asm2 = _block_diag_alpha(a_src2, a_dst2)

    h1, al1 = _tc_mm_alpha(xf, W1, Asm1)
    oraw1, ss1 = _sc_gat_aggregate(h1, al1.T.reshape(8 * N), s2, t2)

    h2, al2 = _tc_epi_mm_alpha(oraw1, ss1.reshape(4, N), b1, g1, be1, W2, Asm2)
    oraw2, ss2 = _sc_gat_aggregate(h2, al2.T.reshape(8 * N), s2, t2)

    on, up = _tc_final(oraw2, ss2.reshape(4, N), b2, g2, be2, fcW, fcb, mask, pos)
    return (on.reshape(64, 40, 2), up.reshape(64, 40, 2))
